# baseline (speedup 1.0000x reference)
import functools

import jax
import jax.numpy as jnp
from jax import lax
from jax.experimental import pallas as pl
from jax.experimental.pallas import tpu as pltpu
from jax.experimental.pallas import tpu_sc as plsc

D = 128
SEQ = 200
BATCH = 1024
NC = 2
NS = 16
NW = NC * NS
ROWS_PER_W = BATCH // NW
SPLIT = 128
NBUF = 4


def _body(idx_hbm, table_hbm, pe_hbm, out_hbm,
          idx_v, buf0, buf1, buf2, buf3, sem0, sem1, sem2, sem3):
    cid = lax.axis_index("c")
    sid = lax.axis_index("s")
    wid = sid * NC + cid

    pltpu.sync_copy(idx_hbm.at[wid], idx_v)

    bufs = (buf0, buf1, buf2, buf3)
    sems = (sem0, sem1, sem2, sem3)
    pe_src = pe_hbm.at[pl.ds(0, SEQ)]

    def prefill(g):
        b = g % NBUF
        return pltpu.async_copy(pe_src, bufs[b], sems[b])

    def gather(g):
        b = g % NBUF
        c0 = pltpu.async_copy(
            table_hbm.at[idx_v.at[g, pl.ds(0, SPLIT)]],
            bufs[b].at[pl.ds(0, SPLIT)], sems[b], add=True)
        c1 = pltpu.async_copy(
            table_hbm.at[idx_v.at[g, pl.ds(SPLIT, SEQ - SPLIT)]],
            bufs[b].at[pl.ds(SPLIT, SEQ - SPLIT)], sems[b], add=True)
        return c0, c1

    def write(g):
        b = g % NBUF
        return pltpu.async_copy(
            bufs[b], out_hbm.at[pl.ds((wid * ROWS_PER_W + g) * SEQ, SEQ)],
            sems[b])

    pres = {g: prefill(g) for g in range(3)}
    pres.pop(0).wait()
    gats = {0: gather(0)}
    writes = {}

    for g in range(ROWS_PER_W):
        for c in gats.pop(g):
            c.wait()
        writes[g] = write(g)
        if g + 1 < ROWS_PER_W:
            pres.pop(g + 1).wait()
            gats[g + 1] = gather(g + 1)
        if g + 3 < ROWS_PER_W:
            if g >= 1:
                writes.pop(g - 1).wait()
            pres[g + 3] = prefill(g + 3)

    for g in sorted(writes):
        writes.pop(g).wait()


_emb = functools.partial(
    pl.kernel,
    out_type=jax.ShapeDtypeStruct((BATCH * SEQ, D), jnp.float32),
    mesh=plsc.VectorSubcoreMesh(core_axis_name="c", subcore_axis_name="s"),
    scratch_types=[
        pltpu.VMEM((ROWS_PER_W, SEQ), jnp.int32),
        pltpu.VMEM((SEQ, D), jnp.float32),
        pltpu.VMEM((SEQ, D), jnp.float32),
        pltpu.VMEM((SEQ, D), jnp.float32),
        pltpu.VMEM((SEQ, D), jnp.float32),
        pltpu.SemaphoreType.DMA,
        pltpu.SemaphoreType.DMA,
        pltpu.SemaphoreType.DMA,
        pltpu.SemaphoreType.DMA,
    ],
)(_body)


@jax.jit
def kernel(inputs, table, pos_encoding):
    idx3 = inputs.astype(jnp.int32).reshape(NW, ROWS_PER_W, SEQ)
    out = _emb(idx3, table, pos_encoding)
    return out.reshape(BATCH, SEQ, D)


# speedup vs baseline: 1.0226x; 1.0226x over previous
import functools

import jax
import jax.numpy as jnp
from jax import lax
from jax.experimental import pallas as pl
from jax.experimental.pallas import tpu as pltpu
from jax.experimental.pallas import tpu_sc as plsc

D = 128
SEQ = 200
BATCH = 1024
NC = 2
NS = 16
NW = NC * NS
GB = 4
GS = 8
SB = SEQ // GS
KB = 8
BBLK = BATCH // GB
CB = BBLK // KB
CHUNK = KB * SB
SPLIT = 128
LANES = 16
NBUF = 3


def _body(idx_hbm, table_hbm, pe_hbm, out_hbm,
          idx_v, pe_v, buf0, buf1, buf2, sem0, sem1, sem2):
    cid = lax.axis_index("c")
    sid = lax.axis_index("s")
    wid = sid * NC + cid
    gb = wid // GS
    gs = wid % GS

    pltpu.sync_copy(idx_hbm.at[wid], idx_v)
    pltpu.sync_copy(pe_hbm.at[gs], pe_v)

    bufs = (buf0, buf1, buf2)
    sems = (sem0, sem1, sem2)

    def start_gather(c):
        b = c % NBUF
        c0 = pltpu.async_copy(
            table_hbm.at[idx_v.at[c, pl.ds(0, SPLIT)]],
            bufs[b].at[pl.ds(0, SPLIT)], sems[b])
        c1 = pltpu.async_copy(
            table_hbm.at[idx_v.at[c, pl.ds(SPLIT, CHUNK - SPLIT)]],
            bufs[b].at[pl.ds(SPLIT, CHUNK - SPLIT)], sems[b])
        return c0, c1

    gathers = {0: start_gather(0), 1: start_gather(1)}
    writes = {}

    for c in range(CB):
        b = c % NBUF
        buf = bufs[b]
        for d in gathers.pop(c):
            d.wait()

        def add_pe(s, carry):
            pv = [pe_v[s, pl.ds(cc * LANES, LANES)] for cc in range(D // LANES)]
            for k in range(KB):
                for cc in range(D // LANES):
                    plsc.addupdate(
                        buf.at[k * SB + s, pl.ds(cc * LANES, LANES)], pv[cc])
            return carry

        lax.fori_loop(0, SB, add_pe, 0)

        batch0 = gb * BBLK + c * KB
        writes[c] = [
            pltpu.async_copy(
                buf.at[pl.ds(k * SB, SB)],
                out_hbm.at[batch0 + k, gs], sems[b])
            for k in range(KB)
        ]

        if c + 2 < CB:
            if c - 1 >= 0:
                for d in writes.pop(c - 1):
                    d.wait()
            gathers[c + 2] = start_gather(c + 2)

    for c in sorted(writes):
        for d in writes.pop(c):
            d.wait()


_emb = functools.partial(
    pl.kernel,
    out_type=jax.ShapeDtypeStruct((BATCH, GS, SB, D), jnp.float32),
    mesh=plsc.VectorSubcoreMesh(core_axis_name="c", subcore_axis_name="s"),
    scratch_types=[
        pltpu.VMEM((CB, CHUNK), jnp.int32),
        pltpu.VMEM((SB, D), jnp.float32),
        pltpu.VMEM((CHUNK, D), jnp.float32),
        pltpu.VMEM((CHUNK, D), jnp.float32),
        pltpu.VMEM((CHUNK, D), jnp.float32),
        pltpu.SemaphoreType.DMA,
        pltpu.SemaphoreType.DMA,
        pltpu.SemaphoreType.DMA,
    ],
)(_body)


@jax.jit
def kernel(inputs, table, pos_encoding):
    idx = inputs.astype(jnp.int32).reshape(GB, CB, KB, GS, SB)
    idx = idx.transpose(0, 3, 1, 2, 4).reshape(NW, CB, CHUNK)
    pe = pos_encoding[:SEQ].reshape(GS, SB, D)
    out = _emb(idx, table, pe)
    return out.reshape(BATCH, SEQ, D)


# speedup vs baseline: 2.7271x; 2.6669x over previous
import functools

import jax
import jax.numpy as jnp
from jax import lax
from jax.experimental import pallas as pl
from jax.experimental.pallas import tpu as pltpu
from jax.experimental.pallas import tpu_sc as plsc

D = 128
SEQ = 200
BATCH = 1024
NC = 2
NS = 16
NW = NC * NS
ROWS_W = BATCH * SEQ // NW
CHUNK = 128
NCH = ROWS_W // CHUNK
LANES = 16
NBUF = 5
LEAD = 3
LAG = 2


def _body(idx_hbm, table_hbm, pe_hbm, out_hbm,
          idx_v, pe_v, buf0, buf1, buf2, buf3, buf4,
          sem0, sem1, sem2, sem3, sem4):
    cid = lax.axis_index("c")
    sid = lax.axis_index("s")
    wid = sid * NC + cid

    pltpu.sync_copy(idx_hbm.at[wid], idx_v)
    pltpu.sync_copy(pe_hbm.at[pl.ds(0, SEQ)], pe_v)

    bufs = (buf0, buf1, buf2, buf3, buf4)
    sems = (sem0, sem1, sem2, sem3, sem4)

    def start_gather(c):
        b = c % NBUF
        return pltpu.async_copy(table_hbm.at[idx_v.at[c]], bufs[b], sems[b])

    gathers = {c: start_gather(c) for c in range(LEAD)}
    writes = {}

    for c in range(NCH):
        b = c % NBUF
        buf = bufs[b]
        gathers.pop(c).wait()

        p0 = (c * CHUNK) % SEQ
        n_first = min(SEQ - p0, CHUNK)

        def seg(lo, hi, pe_off):
            def add_pe(r, carry):
                for cc in range(D // LANES):
                    sl = pl.ds(cc * LANES, LANES)
                    plsc.addupdate(buf.at[r, sl], pe_v[r + pe_off, sl])
                return carry
            lax.fori_loop(lo, hi, add_pe, 0)

        seg(0, n_first, p0)
        if n_first < CHUNK:
            seg(n_first, CHUNK, -n_first)

        writes[c] = pltpu.async_copy(buf, out_hbm.at[wid * NCH + c], sems[b])

        if c + LEAD < NCH:
            if c - LAG >= 0:
                writes.pop(c - LAG).wait()
            gathers[c + LEAD] = start_gather(c + LEAD)

    for c in sorted(writes):
        writes.pop(c).wait()


_emb = functools.partial(
    pl.kernel,
    out_type=jax.ShapeDtypeStruct((NW * NCH, CHUNK, D), jnp.float32),
    mesh=plsc.VectorSubcoreMesh(core_axis_name="c", subcore_axis_name="s"),
    scratch_types=[
        pltpu.VMEM((NCH, CHUNK), jnp.int32),
        pltpu.VMEM((SEQ, D), jnp.float32),
        pltpu.VMEM((CHUNK, D), jnp.float32),
        pltpu.VMEM((CHUNK, D), jnp.float32),
        pltpu.VMEM((CHUNK, D), jnp.float32),
        pltpu.VMEM((CHUNK, D), jnp.float32),
        pltpu.VMEM((CHUNK, D), jnp.float32),
        pltpu.SemaphoreType.DMA,
        pltpu.SemaphoreType.DMA,
        pltpu.SemaphoreType.DMA,
        pltpu.SemaphoreType.DMA,
        pltpu.SemaphoreType.DMA,
    ],
)(_body)


@jax.jit
def kernel(inputs, table, pos_encoding):
    idx = inputs.astype(jnp.int32).reshape(NW, NCH, CHUNK)
    out = _emb(idx, table, pos_encoding)
    return out.reshape(BATCH, SEQ, D)
